# Initial kernel scaffold; baseline (speedup 1.0000x reference)
import jax
import jax.numpy as jnp
from jax.experimental import pallas as pl


def kernel(x, params):
    raise NotImplementedError("write your pallas kernel here")


# speedup vs baseline: 3.5788x; 3.5788x over previous
import functools

import jax
import jax.numpy as jnp
from jax.experimental import pallas as pl

_BIG = 1e30


def _rup(v, m):
    return (v + m - 1) // m * m


def _pad2(a, m, n):
    return jnp.pad(a, ((0, m - a.shape[0]), (0, n - a.shape[1])))


def _mm_kernel(x_ref, w_ref, s_ref, o_ref, *, gelu):
    acc = jnp.dot(x_ref[...], w_ref[...], preferred_element_type=jnp.float32)
    y = acc * s_ref[0:1, :] + s_ref[1:2, :]
    if gelu:
        y = jax.nn.gelu(y)
    o_ref[...] = y


def _mm_res_kernel(x_ref, w_ref, s_ref, r_ref, o_ref, *, gelu):
    acc = jnp.dot(x_ref[...], w_ref[...], preferred_element_type=jnp.float32)
    y = acc * s_ref[0:1, :] + s_ref[1:2, :]
    if gelu:
        y = jax.nn.gelu(y)
    o_ref[...] = r_ref[...] + y


def _mm2_kernel(a_ref, b_ref, wa_ref, wb_ref, s_ref, o_ref, *, gelu):
    acc = jnp.dot(a_ref[...], wa_ref[...], preferred_element_type=jnp.float32)
    acc = acc + jnp.dot(b_ref[...], wb_ref[...],
                        preferred_element_type=jnp.float32)
    y = acc * s_ref[0:1, :] + s_ref[1:2, :]
    if gelu:
        y = jax.nn.gelu(y)
    o_ref[...] = y


def matmul2_affine(a, b, wa, wb, scale, shift, *, gelu=False):
    m, ka = a.shape
    kb = b.shape[1]
    nd = wa.shape[1]
    mp, kap, kbp, np_ = _rup(m, 8), _rup(ka, 128), _rup(kb, 128), _rup(nd, 128)
    s = jnp.zeros((8, np_), jnp.float32)
    s = s.at[0, :nd].set(scale).at[1, :nd].set(shift)
    out = pl.pallas_call(
        functools.partial(_mm2_kernel, gelu=gelu),
        out_shape=jax.ShapeDtypeStruct((mp, np_), jnp.float32),
    )(_pad2(a, mp, kap), _pad2(b, mp, kbp),
      _pad2(wa, kap, np_), _pad2(wb, kbp, np_), s)
    return out[:m, :nd]


def matmul_affine(x, w, scale, shift, *, gelu=False, residual=None):
    m, kd = x.shape
    _, nd = w.shape
    mp, kp, np_ = _rup(m, 8), _rup(kd, 128), _rup(nd, 128)
    xp = _pad2(x, mp, kp)
    wp = _pad2(w, kp, np_)
    s = jnp.zeros((8, np_), jnp.float32)
    s = s.at[0, :nd].set(scale).at[1, :nd].set(shift)
    args = [xp, wp, s]
    if residual is not None:
        body = functools.partial(_mm_res_kernel, gelu=gelu)
        args.append(_pad2(residual, mp, np_))
    else:
        body = functools.partial(_mm_kernel, gelu=gelu)
    out = pl.pallas_call(
        body,
        out_shape=jax.ShapeDtypeStruct((mp, np_), jnp.float32),
    )(*args)
    return out[:m, :nd]


def _knn_kernel(frow_ref, fall_ref, o_ref, *, nreal, k):
    f_rows = frow_ref[...]
    f_all = fall_ref[...]
    t, np_ = f_rows.shape[0], f_all.shape[0]
    sq_rows = jnp.sum(f_rows * f_rows, axis=1, keepdims=True)
    ones = jnp.ones((1, f_all.shape[1]), jnp.float32)
    sq_all_t = jax.lax.dot_general(
        ones, f_all * f_all, (((1,), (1,)), ((), ())),
        precision=jax.lax.Precision.HIGHEST,
        preferred_element_type=jnp.float32)
    cross = jax.lax.dot_general(
        f_rows, f_all, (((1,), (1,)), ((), ())),
        preferred_element_type=jnp.float32)
    dist = sq_rows + sq_all_t - 2.0 * cross
    iota = jax.lax.broadcasted_iota(jnp.int32, (t, np_), 1)
    dist = jnp.where(iota < nreal, dist, _BIG)

    mx = jnp.full((t, f_all.shape[1]), -_BIG, jnp.float32)
    s = dist
    for _ in range(k):
        m = jnp.min(s, axis=1, keepdims=True)
        ismin = s <= m
        idx = jnp.min(jnp.where(ismin, iota, np_), axis=1, keepdims=True)
        onehot = iota == idx
        g = jax.lax.dot_general(
            onehot.astype(jnp.float32), f_all, (((1,), (0,)), ((), ())),
            precision=jax.lax.Precision.HIGHEST,
            preferred_element_type=jnp.float32)
        mx = jnp.maximum(mx, g)
        s = jnp.where(onehot, _BIG, s)
    o_ref[...] = mx - f_rows


def knn_maxagg(f, nreal, tile, k=9):
    n, c = f.shape
    np_ = _rup(_rup(n, tile), 128)
    cp = _rup(c, 128)
    fp = _pad2(f, np_, cp)
    grid = np_ // tile
    out = pl.pallas_call(
        functools.partial(_knn_kernel, nreal=n, k=k),
        grid=(grid,),
        in_specs=[
            pl.BlockSpec((tile, cp), lambda i: (i, 0)),
            pl.BlockSpec((np_, cp), lambda i: (0, 0)),
        ],
        out_specs=pl.BlockSpec((tile, cp), lambda i: (i, 0)),
        out_shape=jax.ShapeDtypeStruct((np_, cp), jnp.float32),
    )(fp, fp)
    return out[:n, :c]


def _pool_fc_kernel(x_ref, w_ref, o_ref, *, nreal):
    s = jnp.sum(x_ref[...], axis=0, keepdims=True) / nreal
    o_ref[...] = jnp.dot(s, w_ref[...], preferred_element_type=jnp.float32)


def pool_fc(f, w):
    n, c = f.shape
    np_, cp = _rup(n, 8), _rup(c, 128)
    fp = _pad2(f, np_, cp)
    wp = _pad2(w, cp, 128)
    out = pl.pallas_call(
        functools.partial(_pool_fc_kernel, nreal=n),
        out_shape=jax.ShapeDtypeStruct((1, 128), jnp.float32),
    )(fp, wp)
    return out[:1, :1]


def conv3x3(x, p, *, stride, gelu=False):
    y = jax.lax.conv_general_dilated(
        x, p['w'], (stride, stride), 'SAME',
        dimension_numbers=('NHWC', 'HWIO', 'NHWC')) + p['b']
    y = y * p['g'] + p['bt']
    if gelu:
        y = jax.nn.gelu(y)
    return y


_STAGE_TILE = {0: 256, 1: 1024, 2: 256, 3: 128}


def _aff(p, prefix):
    g = p[prefix + '_g'] if prefix else p['g']
    b = p[prefix + '_b'] if prefix else p['b']
    bt = p[prefix + '_bt'] if prefix else p['bt']
    return g, b * g + bt


def _vig_block(f, bp, tile):
    n, c = f.shape
    sc, sh = _aff(bp, 'fc1')
    y = matmul_affine(f, bp['fc1_w'].reshape(c, c), sc, sh)
    mx = knn_maxagg(y, n, tile)
    sc, sh = _aff(bp, 'g')
    gw = bp['g_w'].reshape(2 * c, 2 * c)
    g = matmul2_affine(y, mx, gw[:c], gw[c:], sc, sh, gelu=True)
    sc, sh = _aff(bp, 'fc2')
    f = matmul_affine(g, bp['fc2_w'].reshape(2 * c, c), sc, sh, residual=f)
    sc, sh = _aff(bp, 'ffn1')
    hid = matmul_affine(f, bp['ffn1_w'].reshape(c, 4 * c), sc, sh, gelu=True)
    sc, sh = _aff(bp, 'ffn2')
    f = matmul_affine(hid, bp['ffn2_w'].reshape(4 * c, c), sc, sh, residual=f)
    return f


def kernel(x, params):
    img = jnp.transpose(x, (0, 2, 3, 1))
    s = params['stem']
    h = conv3x3(img, s[0], stride=2, gelu=True)
    h = conv3x3(h, s[1], stride=2, gelu=True)
    h = conv3x3(h, s[2], stride=1)

    hw = 56
    f = h.reshape(hw * hw, h.shape[-1])
    for si in range(4):
        tile = _STAGE_TILE[si]
        for bp in params['stages'][si]:
            f = _vig_block(f, bp, tile)
        if si < 3:
            h = conv3x3(f.reshape(1, hw, hw, f.shape[-1]), params['downs'][si],
                        stride=2)
            hw //= 2
            f = h.reshape(hw * hw, h.shape[-1])

    out = pool_fc(f, params['fc_w'])
    return out + params['fc_b']
